# baseline (speedup 1.0000x reference)
import jax
import jax.numpy as jnp
from jax.experimental import pallas as pl

_EPS = 1e-8


def _simrel_kernel(x_ref, ca_ref, out_ref):
    b, s, d = x_ref.shape
    k = ca_ref.shape[0]
    x = x_ref[...].reshape(b * s, d)
    ca = ca_ref[...]
    inv_in = 1.0 / jnp.maximum(jnp.sqrt(jnp.sum(x * x, axis=1, keepdims=True)), _EPS)
    inv_ca = 1.0 / jnp.maximum(jnp.sqrt(jnp.sum(ca * ca, axis=1)), _EPS)
    dots = jax.lax.dot_general(
        x, ca,
        dimension_numbers=(((1,), (1,)), ((), ())),
        preferred_element_type=jnp.float32,
    )
    out_ref[...] = (dots * inv_in * inv_ca[None, :]).reshape(b, s, k)


def kernel(inputs, labels, class_avgs):
    del labels
    b, s, d = inputs.shape
    k = class_avgs.shape[0]
    return pl.pallas_call(
        _simrel_kernel,
        out_shape=jax.ShapeDtypeStruct((b, s, k), jnp.float32),
    )(inputs, class_avgs)


# speedup vs baseline: 1.7005x; 1.7005x over previous
import jax
import jax.numpy as jnp
from jax.experimental import pallas as pl

_EPS = 1e-8


def _simrel_kernel(x_ref, ca_ref, out_ref):
    b = x_ref.shape[0]
    ca = ca_ref[...]
    inv_ca = 1.0 / jnp.maximum(jnp.sqrt(jnp.sum(ca * ca, axis=1, keepdims=True)), _EPS)
    for i in range(b):
        x = x_ref[i]
        inv_in = 1.0 / jnp.maximum(jnp.sqrt(jnp.sum(x * x, axis=1)), _EPS)
        dots = jax.lax.dot_general(
            ca, x,
            dimension_numbers=(((1,), (1,)), ((), ())),
            preferred_element_type=jnp.float32,
        )
        out_ref[i] = dots * inv_ca * inv_in[None, :]


def kernel(inputs, labels, class_avgs):
    del labels
    b, s, d = inputs.shape
    k = class_avgs.shape[0]
    out_t = pl.pallas_call(
        _simrel_kernel,
        out_shape=jax.ShapeDtypeStruct((b, k, s), jnp.float32),
    )(inputs, class_avgs)
    return jnp.swapaxes(out_t, 1, 2)
